# Initial kernel scaffold; baseline (speedup 1.0000x reference)
import jax
import jax.numpy as jnp
from jax.experimental import pallas as pl


def kernel(y, start_nodes, end_nodes, degree_factors_start, degree_factors, W_h0, b_h0, W_out, b_out):
    raise NotImplementedError("write your pallas kernel here")


# speedup vs baseline: 3.0621x; 3.0621x over previous
import functools
import jax
import jax.numpy as jnp
from jax import lax
from jax.experimental import pallas as pl
from jax.experimental.pallas import tpu as pltpu
from jax.experimental.pallas import tpu_sc as plsc

_N = 10000
_E = 320000
_D = 128
_NC = 2
_NS = 16
_NW = _NC * _NS
_C = 128
_EPT = _E // _NW
_EPTP = 10240
_NCHUNK = _EPTP // _C
_RPT = 624


def _sc_accumulate(y, start3, end3, dfs3):
    mesh = plsc.VectorSubcoreMesh(core_axis_name="c", subcore_axis_name="s")

    @functools.partial(
        pl.kernel,
        out_type=jax.ShapeDtypeStruct((_NC, _N, _D), jnp.float32),
        mesh=mesh,
        scratch_types=[
            pltpu.VMEM_SHARED((_N, _D), jnp.float32),
            pltpu.VMEM((_C,), jnp.int32),
            pltpu.VMEM((_C,), jnp.int32),
            pltpu.VMEM((_C,), jnp.float32),
            pltpu.VMEM((_C, _D), jnp.float32),
            pltpu.SemaphoreType.DMA,
        ],
    )
    def k(y_hbm, s_hbm, e_hbm, f_hbm, out_hbm, acc, idx_s, idx_e, dfs_c,
          rows, sem):
        cid = lax.axis_index("c")
        sid = lax.axis_index("s")
        wid = sid * _NC + cid

        def zero_row(r, _):
            for kk in range(_D // 16):
                rows[r, pl.ds(kk * 16, 16)] = jnp.zeros((16,), jnp.float32)
            return 0

        lax.fori_loop(0, _C, zero_row, 0)
        base0 = sid * _RPT
        off = 0
        while off < _RPT:
            n = min(_C, _RPT - off)
            pltpu.sync_copy(rows.at[pl.ds(0, n)],
                            acc.at[pl.ds(base0 + off, n)])
            off += n

        @pl.when(sid == _NS - 1)
        def _zero_tail():
            pltpu.sync_copy(rows.at[pl.ds(0, _N - _NS * _RPT)],
                            acc.at[pl.ds(_NS * _RPT, _N - _NS * _RPT)])

        plsc.subcore_barrier()

        def chunk(j, _):
            pltpu.sync_copy(s_hbm.at[wid, j], idx_s)
            pltpu.sync_copy(e_hbm.at[wid, j], idx_e)
            pltpu.sync_copy(f_hbm.at[wid, j], dfs_c)
            pltpu.async_copy(y_hbm.at[idx_s], rows, sem).wait()

            def scale_group(g, _):
                fv = dfs_c[pl.ds(g * 16, 16)]
                for r16 in range(16):
                    sv = jnp.full((16,), fv[r16], jnp.float32)
                    r = g * 16 + r16
                    for kk in range(_D // 16):
                        sl = pl.ds(kk * 16, 16)
                        rows[r, sl] = rows[r, sl] * sv
                return 0

            lax.fori_loop(0, _C // 16, scale_group, 0)
            pltpu.sync_copy(rows, acc.at[idx_e], add=True)
            return 0

        lax.fori_loop(0, _NCHUNK, chunk, 0)
        plsc.subcore_barrier()

        base = sid * _RPT
        pltpu.sync_copy(acc.at[pl.ds(base, _RPT)],
                        out_hbm.at[cid, pl.ds(base, _RPT)])

        @pl.when(sid == _NS - 1)
        def _publish_tail():
            t = _NS * _RPT
            pltpu.sync_copy(acc.at[pl.ds(t, _N - t)],
                            out_hbm.at[cid, pl.ds(t, _N - t)])

    return k(y, start3, end3, dfs3)


_BT = 1000


def _tc_body(p_ref, df_ref, wh_ref, bh_ref, wo_ref, bo_ref, out_ref):
    agg = p_ref[0] + p_ref[1]
    h = agg * df_ref[...]
    h = lax.dot_general(h, wh_ref[...], (((1,), (1,)), ((), ())),
                        preferred_element_type=jnp.float32)
    h = jnp.maximum(h + bh_ref[...].reshape(1, _D), 0.0)
    o = lax.dot_general(h, wo_ref[...], (((1,), (1,)), ((), ())),
                        preferred_element_type=jnp.float32)
    out_ref[...] = o + bo_ref[...].reshape(1, _D)


def _tc_mlp(partials, degree_factors, W_h0, b_h0, W_out, b_out):
    grid = _N // _BT
    return pl.pallas_call(
        _tc_body,
        grid=(grid,),
        in_specs=[
            pl.BlockSpec((_NC, _BT, _D), lambda i: (0, i, 0)),
            pl.BlockSpec((_BT, 1), lambda i: (i, 0)),
            pl.BlockSpec((_D, _D), lambda i: (0, 0)),
            pl.BlockSpec((_D,), lambda i: (0,)),
            pl.BlockSpec((_D, _D), lambda i: (0, 0)),
            pl.BlockSpec((_D,), lambda i: (0,)),
        ],
        out_specs=pl.BlockSpec((_BT, _D), lambda i: (i, 0)),
        out_shape=jax.ShapeDtypeStruct((_N, _D), jnp.float32),
    )(partials, degree_factors, W_h0, b_h0, W_out, b_out)


def _pad_tiles(x, fill):
    x = x.reshape(_NW, _EPT)
    pad = jnp.full((_NW, _EPTP - _EPT), fill, x.dtype)
    return jnp.concatenate([x, pad], axis=1).reshape(_NW, _NCHUNK, _C)


@jax.jit
def kernel(y, start_nodes, end_nodes, degree_factors_start, degree_factors,
           W_h0, b_h0, W_out, b_out):
    start3 = _pad_tiles(start_nodes.astype(jnp.int32), 0)
    end3 = _pad_tiles(end_nodes.astype(jnp.int32), 0)
    dfs3 = _pad_tiles(degree_factors_start.reshape(_E), 0.0)
    partials = _sc_accumulate(y, start3, end3, dfs3)
    out = _tc_mlp(partials, degree_factors, W_h0, b_h0, W_out, b_out)
    return (out, start_nodes, end_nodes, degree_factors_start, degree_factors)
